# Initial kernel scaffold; baseline (speedup 1.0000x reference)
import jax
import jax.numpy as jnp
from jax.experimental import pallas as pl


def kernel(node_features, edge_index, W_embed, b_embed, W_e, b_e, W_h, b_h, W_x):
    raise NotImplementedError("write your pallas kernel here")


# speedup vs baseline: 5.4446x; 5.4446x over previous
import functools

import jax
import jax.numpy as jnp
from jax import lax
from jax.experimental import pallas as pl
from jax.experimental.pallas import tpu as pltpu
from jax.experimental.pallas import tpu_sc as plsc

_N = 10000
_H = 128
_HID = 128
_E = 320000
_NCLS = 8

_NCORES = 2
_NSUB = 16
_NW = _NCORES * _NSUB
_EPW = _E // _NW
_CH = 80
_NCHUNK = _EPW // _CH
_TW = 144


def _tables_body(nf_ref, we_ref, be_ref, we1_ref, we2_ref, bee_ref,
                 ta_ref, tb_ref):
    h = nf_ref[:, :_H]
    x = nf_ref[:, _H:_H + 2]
    bf = jnp.bfloat16
    h_emb = jnp.dot(h.astype(bf), we_ref[...].astype(bf),
                    preferred_element_type=jnp.float32)
    h_emb = h_emb + be_ref[...][None, :]
    he16 = h_emb.astype(bf)
    a = jnp.dot(he16, we1_ref[...].astype(bf),
                preferred_element_type=jnp.float32)
    b = jnp.dot(he16, we2_ref[...].astype(bf),
                preferred_element_type=jnp.float32)
    b = b + bee_ref[...][None, :]
    pad = jnp.zeros((h.shape[0], _TW - _H - 2), jnp.float32)
    ta_ref[...] = jnp.concatenate([a, x, pad], axis=1)
    tb_ref[...] = jnp.concatenate([b, x, pad], axis=1)


def _build_tables(node_features, W_embed, b_embed, We1, We2, b_e):
    blk = 2000
    grid = _N // blk
    return pl.pallas_call(
        _tables_body,
        grid=(grid,),
        in_specs=[
            pl.BlockSpec((blk, _H + 2), lambda i: (i, 0)),
            pl.BlockSpec((_H, _HID), lambda i: (0, 0)),
            pl.BlockSpec((_HID,), lambda i: (0,)),
            pl.BlockSpec((_HID, _HID), lambda i: (0, 0)),
            pl.BlockSpec((_HID, _HID), lambda i: (0, 0)),
            pl.BlockSpec((_HID,), lambda i: (0,)),
        ],
        out_specs=[
            pl.BlockSpec((blk, _TW), lambda i: (i, 0)),
            pl.BlockSpec((blk, _TW), lambda i: (i, 0)),
        ],
        out_shape=[
            jax.ShapeDtypeStruct((_N, _TW), jnp.float32),
            jax.ShapeDtypeStruct((_N, _TW), jnp.float32),
        ],
    )(node_features, W_embed, b_embed, We1, We2, b_e)


def _edge_body(ta_hbm, tb_hbm, src_hbm, dst_hbm, w_hbm, wx_hbm,
               out_hbm,
               srcb, dstb, ra, rb, vacc, wv, wxv, semA, semB):
    cid = lax.axis_index("c")
    sid = lax.axis_index("s")
    wid = sid * _NCORES + cid

    def bf16round(x):
        i = lax.bitcast_convert_type(x, jnp.int32)
        r = i + 0x7FFF + ((i >> 16) & 1)
        r = r & jnp.int32(-65536)
        return lax.bitcast_convert_type(r, jnp.float32)

    iota = lax.broadcasted_iota(jnp.int32, (16,), 0)
    zero16 = jnp.zeros((16,), jnp.float32)
    pltpu.sync_copy(w_hbm, wv)
    pltpu.sync_copy(wx_hbm, wxv)
    wjs = [bf16round(wv[pl.ds(16 * j, 16)]) for j in range(8)]
    wxjs = [bf16round(wxv[pl.ds(16 * j, 16)]) for j in range(8)]

    def zero_body(k, carry):
        vacc[pl.ds(k * 16, 16)] = zero16
        return carry

    lax.fori_loop(0, (2 * _N) // 16, zero_body, 0)

    def take16(vec, idx):
        return jnp.take_along_axis(vec, idx, axis=0, mode="promise_in_bounds")

    base = wid * _EPW

    def chunk_body(ci, carry):
        off = base + ci * _CH
        pltpu.sync_copy(src_hbm.at[pl.ds(off, _CH)], srcb)
        pltpu.sync_copy(dst_hbm.at[pl.ds(off, _CH)], dstb)
        cpa = pltpu.async_copy(ta_hbm.at[srcb], ra, semA)
        cpb = pltpu.async_copy(tb_hbm.at[dstb], rb, semB)
        cpa.wait()
        cpb.wait()

        def group_body(g, gcarry):
            rows = g * 16 + iota
            c0 = jnp.full((16,), _H, jnp.int32)
            c1 = jnp.full((16,), _H + 1, jnp.int32)
            xs0 = plsc.load_gather(ra, [rows, c0])
            xs1 = plsc.load_gather(ra, [rows, c1])
            xd0 = plsc.load_gather(rb, [rows, c0])
            xd1 = plsc.load_gather(rb, [rows, c1])
            dstv = dstb[pl.ds(g * 16, 16)]
            d0 = xs0 - xd0
            d1 = xs1 - xd1
            d2 = bf16round(d0 * d0 + d1 * d1)
            for l in range(16):
                e = g * 16 + l
                lidx = jnp.full((16,), l, jnp.int32)
                dist2 = take16(d2, lidx)
                acc = jnp.zeros((16,), jnp.float32)
                for j in range(8):
                    av = ra[e, pl.ds(16 * j, 16)]
                    bv = rb[e, pl.ds(16 * j, 16)]
                    m = jnp.maximum(bf16round(av + bv + dist2 * wjs[j]), 0.0)
                    acc = acc + m * wxjs[j]
                for sh in (8, 4, 2, 1):
                    acc = acc + take16(acc, iota ^ sh)
                dvec = jnp.where(iota == 0, take16(d0, lidx), take16(d1, lidx))
                didx = take16(dstv, lidx) + iota * _N
                plsc.addupdate_scatter(vacc, [didx], acc * dvec,
                                       mask=iota < 2)
            return gcarry

        lax.fori_loop(0, _CH // 16, group_body, 0)
        return carry

    lax.fori_loop(0, _NCHUNK, chunk_body, 0)

    pltpu.sync_copy(vacc, out_hbm.at[wid])


def _edge_phase(ta, tb, src, dst, w_row, wx):
    mesh = plsc.VectorSubcoreMesh(core_axis_name="c", subcore_axis_name="s")
    f = pl.kernel(
        _edge_body,
        out_type=jax.ShapeDtypeStruct((_NW, 2 * _N), jnp.float32),
        mesh=mesh,
        scratch_types=[
            pltpu.VMEM((_CH,), jnp.int32),
            pltpu.VMEM((_CH,), jnp.int32),
            pltpu.VMEM((_CH, _TW), jnp.float32),
            pltpu.VMEM((_CH, _TW), jnp.float32),
            pltpu.VMEM((2 * _N,), jnp.float32),
            pltpu.VMEM((_HID,), jnp.float32),
            pltpu.VMEM((_HID,), jnp.float32),
            pltpu.SemaphoreType.DMA,
            pltpu.SemaphoreType.DMA,
        ],
        compiler_params=pltpu.CompilerParams(
            use_tc_tiling_on_sc=False, needs_layout_passes=False),
    )
    return f(ta, tb, src, dst, w_row, wx)


def _logits_body(p_ref, out_ref):
    v = jnp.sum(p_ref[...], axis=0)
    x = v[0:1, :]
    y = v[1:2, :]
    nx = -x
    c1 = (y > 0) & (y >= x) & (x > 0)
    c2 = (y > 0) & (x <= 0) & (y > nx)
    c3 = (y > 0) & (x < 0) & (y <= nx)
    c4 = (y <= 0) & (x < 0) & (y > x)
    c5 = (y < 0) & (x < 0) & (y <= x)
    c6 = (y < 0) & (x >= 0) & (-y > x)
    c7 = (y < 0) & (x > 0) & (-y <= x)
    cls = (c1.astype(jnp.int32) + 2 * c2.astype(jnp.int32)
           + 3 * c3.astype(jnp.int32) + 4 * c4.astype(jnp.int32)
           + 5 * c5.astype(jnp.int32) + 6 * c6.astype(jnp.int32)
           + 7 * c7.astype(jnp.int32))
    k = lax.broadcasted_iota(jnp.int32, (_NCLS, _N), 0)
    out_ref[...] = jnp.where(cls == k, 0.0, -1000.0)


def _make_logits(partials):
    return pl.pallas_call(
        _logits_body,
        out_shape=jax.ShapeDtypeStruct((_NCLS, _N), jnp.float32),
    )(partials)


def kernel(node_features, edge_index, W_embed, b_embed, W_e, b_e, W_h, b_h,
           W_x):
    del W_h, b_h
    We1 = W_e[:_HID]
    We2 = W_e[_HID:2 * _HID]
    w_row = W_e[2 * _HID]
    wx = W_x[:, 0]
    ta, tb = _build_tables(node_features, W_embed, b_embed, We1, We2, b_e)
    src = edge_index[0]
    dst = edge_index[1]
    partials = _edge_phase(ta, tb, src, dst, w_row, wx)
    logits_t = _make_logits(partials.reshape(_NW, 2, _N))
    return logits_t.T


# speedup vs baseline: 8.1086x; 1.4893x over previous
import functools

import jax
import jax.numpy as jnp
from jax import lax
from jax.experimental import pallas as pl
from jax.experimental.pallas import tpu as pltpu
from jax.experimental.pallas import tpu_sc as plsc

_N = 10000
_H = 128
_HID = 128
_E = 320000
_NCLS = 8

_NCORES = 2
_NSUB = 16
_NW = _NCORES * _NSUB
_EPW = _E // _NW
_CH = 80
_NCHUNK = _EPW // _CH
_TW = 144


def _tables_body(nf_ref, we_ref, be_ref, we1_ref, we2_ref, bee_ref,
                 ta_ref, tb_ref):
    h = nf_ref[:, :_H]
    x = nf_ref[:, _H:_H + 2]
    bf = jnp.bfloat16
    h_emb = jnp.dot(h.astype(bf), we_ref[...].astype(bf),
                    preferred_element_type=jnp.float32)
    h_emb = h_emb + be_ref[...][None, :]
    he16 = h_emb.astype(bf)
    a = jnp.dot(he16, we1_ref[...].astype(bf),
                preferred_element_type=jnp.float32)
    b = jnp.dot(he16, we2_ref[...].astype(bf),
                preferred_element_type=jnp.float32)
    b = b + bee_ref[...][None, :]
    pad = jnp.zeros((h.shape[0], _TW - _H - 2), jnp.float32)
    ta_ref[...] = jnp.concatenate([a, x, pad], axis=1)
    tb_ref[...] = jnp.concatenate([b, x, pad], axis=1)


def _build_tables(node_features, W_embed, b_embed, We1, We2, b_e):
    blk = 2000
    grid = _N // blk
    return pl.pallas_call(
        _tables_body,
        grid=(grid,),
        in_specs=[
            pl.BlockSpec((blk, _H + 2), lambda i: (i, 0)),
            pl.BlockSpec((_H, _HID), lambda i: (0, 0)),
            pl.BlockSpec((_HID,), lambda i: (0,)),
            pl.BlockSpec((_HID, _HID), lambda i: (0, 0)),
            pl.BlockSpec((_HID, _HID), lambda i: (0, 0)),
            pl.BlockSpec((_HID,), lambda i: (0,)),
        ],
        out_specs=[
            pl.BlockSpec((blk, _TW), lambda i: (i, 0)),
            pl.BlockSpec((blk, _TW), lambda i: (i, 0)),
        ],
        out_shape=[
            jax.ShapeDtypeStruct((_N, _TW), jnp.float32),
            jax.ShapeDtypeStruct((_N, _TW), jnp.float32),
        ],
    )(node_features, W_embed, b_embed, We1, We2, b_e)


def _edge_body(ta_hbm, tb_hbm, src_hbm, dst_hbm, w_hbm, wx_hbm,
               out_hbm,
               sall, dall, ra0, rb0, ra1, rb1, vacc, wv, wxv,
               sA0, sB0, sA1, sB1):
    cid = lax.axis_index("c")
    sid = lax.axis_index("s")
    wid = sid * _NCORES + cid

    def bf16round(x):
        i = lax.bitcast_convert_type(x, jnp.int32)
        r = i + 0x7FFF + ((i >> 16) & 1)
        r = r & jnp.int32(-65536)
        return lax.bitcast_convert_type(r, jnp.float32)

    iota = lax.broadcasted_iota(jnp.int32, (16,), 0)
    zero16 = jnp.zeros((16,), jnp.float32)
    pltpu.sync_copy(w_hbm, wv)
    pltpu.sync_copy(wx_hbm, wxv)
    wjs = [bf16round(wv[pl.ds(16 * j, 16)]) for j in range(8)]
    wxjs = [bf16round(wxv[pl.ds(16 * j, 16)]) for j in range(8)]

    def zero_body(k, carry):
        vacc[pl.ds(k * 16, 16)] = zero16
        return carry

    lax.fori_loop(0, (2 * _N) // 16, zero_body, 0)

    def take16(vec, idx):
        return jnp.take_along_axis(vec, idx, axis=0, mode="promise_in_bounds")

    base = wid * _EPW

    pltpu.sync_copy(src_hbm.at[pl.ds(base, _EPW)], sall)
    pltpu.sync_copy(dst_hbm.at[pl.ds(base, _EPW)], dall)

    def issue(ci, ra, rb, sA, sB):
        pltpu.async_copy(ta_hbm.at[sall.at[pl.ds(ci * _CH, _CH)]], ra, sA)
        pltpu.async_copy(tb_hbm.at[dall.at[pl.ds(ci * _CH, _CH)]], rb, sB)

    def wait(ra, rb, sA, sB):
        pltpu.make_async_copy(ta_hbm.at[pl.ds(0, _CH)], ra, sA).wait()
        pltpu.make_async_copy(tb_hbm.at[pl.ds(0, _CH)], rb, sB).wait()

    def compute(ci, ra, rb):
        def group_body(g, gcarry):
            rows = g * 16 + iota
            c0 = jnp.full((16,), _H, jnp.int32)
            c1 = jnp.full((16,), _H + 1, jnp.int32)
            xs0 = plsc.load_gather(ra, [rows, c0])
            xs1 = plsc.load_gather(ra, [rows, c1])
            xd0 = plsc.load_gather(rb, [rows, c0])
            xd1 = plsc.load_gather(rb, [rows, c1])
            dstv = dall[pl.ds(ci * _CH + g * 16, 16)]
            d0 = xs0 - xd0
            d1 = xs1 - xd1
            d2 = bf16round(d0 * d0 + d1 * d1)
            for l in range(16):
                e = g * 16 + l
                lidx = jnp.full((16,), l, jnp.int32)
                dist2 = take16(d2, lidx)
                acc = jnp.zeros((16,), jnp.float32)
                for j in range(8):
                    av = ra[e, pl.ds(16 * j, 16)]
                    bv = rb[e, pl.ds(16 * j, 16)]
                    m = jnp.maximum(bf16round(av + bv + dist2 * wjs[j]), 0.0)
                    acc = acc + m * wxjs[j]
                for sh in (8, 4, 2, 1):
                    acc = acc + take16(acc, iota ^ sh)
                dvec = jnp.where(iota == 0, take16(d0, lidx), take16(d1, lidx))
                didx = take16(dstv, lidx) + iota * _N
                plsc.addupdate_scatter(vacc, [didx], acc * dvec,
                                       mask=iota < 2)
            return gcarry

        lax.fori_loop(0, _CH // 16, group_body, 0)

    issue(0, ra0, rb0, sA0, sB0)

    def pair_body(k, carry):
        c0 = 2 * k
        wait(ra0, rb0, sA0, sB0)
        issue(c0 + 1, ra1, rb1, sA1, sB1)
        compute(c0, ra0, rb0)
        wait(ra1, rb1, sA1, sB1)
        issue(c0 + 2, ra0, rb0, sA0, sB0)
        compute(c0 + 1, ra1, rb1)
        return carry

    lax.fori_loop(0, (_NCHUNK - 1) // 2, pair_body, 0)
    wait(ra0, rb0, sA0, sB0)
    compute(_NCHUNK - 1, ra0, rb0)

    pltpu.sync_copy(vacc, out_hbm.at[wid])


def _edge_phase(ta, tb, src, dst, w_row, wx):
    mesh = plsc.VectorSubcoreMesh(core_axis_name="c", subcore_axis_name="s")
    f = pl.kernel(
        _edge_body,
        out_type=jax.ShapeDtypeStruct((_NW, 2 * _N), jnp.float32),
        mesh=mesh,
        scratch_types=[
            pltpu.VMEM((_EPW,), jnp.int32),
            pltpu.VMEM((_EPW,), jnp.int32),
            pltpu.VMEM((_CH, _TW), jnp.float32),
            pltpu.VMEM((_CH, _TW), jnp.float32),
            pltpu.VMEM((_CH, _TW), jnp.float32),
            pltpu.VMEM((_CH, _TW), jnp.float32),
            pltpu.VMEM((2 * _N,), jnp.float32),
            pltpu.VMEM((_HID,), jnp.float32),
            pltpu.VMEM((_HID,), jnp.float32),
            pltpu.SemaphoreType.DMA,
            pltpu.SemaphoreType.DMA,
            pltpu.SemaphoreType.DMA,
            pltpu.SemaphoreType.DMA,
        ],
        compiler_params=pltpu.CompilerParams(
            use_tc_tiling_on_sc=False, needs_layout_passes=False),
    )
    return f(ta, tb, src, dst, w_row, wx)


def _logits_body(p_ref, out_ref):
    v = jnp.sum(p_ref[...], axis=0)
    x = v[0:1, :]
    y = v[1:2, :]
    nx = -x
    c1 = (y > 0) & (y >= x) & (x > 0)
    c2 = (y > 0) & (x <= 0) & (y > nx)
    c3 = (y > 0) & (x < 0) & (y <= nx)
    c4 = (y <= 0) & (x < 0) & (y > x)
    c5 = (y < 0) & (x < 0) & (y <= x)
    c6 = (y < 0) & (x >= 0) & (-y > x)
    c7 = (y < 0) & (x > 0) & (-y <= x)
    cls = (c1.astype(jnp.int32) + 2 * c2.astype(jnp.int32)
           + 3 * c3.astype(jnp.int32) + 4 * c4.astype(jnp.int32)
           + 5 * c5.astype(jnp.int32) + 6 * c6.astype(jnp.int32)
           + 7 * c7.astype(jnp.int32))
    k = lax.broadcasted_iota(jnp.int32, (_NCLS, _N), 0)
    out_ref[...] = jnp.where(cls == k, 0.0, -1000.0)


def _make_logits(partials):
    return pl.pallas_call(
        _logits_body,
        out_shape=jax.ShapeDtypeStruct((_NCLS, _N), jnp.float32),
    )(partials)


def kernel(node_features, edge_index, W_embed, b_embed, W_e, b_e, W_h, b_h,
           W_x):
    del W_h, b_h
    We1 = W_e[:_HID]
    We2 = W_e[_HID:2 * _HID]
    w_row = W_e[2 * _HID]
    wx = W_x[:, 0]
    ta, tb = _build_tables(node_features, W_embed, b_embed, We1, We2, b_e)
    src = edge_index[0]
    dst = edge_index[1]
    partials = _edge_phase(ta, tb, src, dst, w_row, wx)
    logits_t = _make_logits(partials.reshape(_NW, 2, _N))
    return logits_t.T


# speedup vs baseline: 8.1114x; 1.0003x over previous
import functools

import jax
import jax.numpy as jnp
from jax import lax
from jax.experimental import pallas as pl
from jax.experimental.pallas import tpu as pltpu
from jax.experimental.pallas import tpu_sc as plsc

_N = 10000
_H = 128
_HID = 128
_E = 320000
_NCLS = 8

_NCORES = 2
_NSUB = 16
_NW = _NCORES * _NSUB
_EPW = _E // _NW
_CH = 80
_NCHUNK = _EPW // _CH
_TW = 144


def _tables_body(nf_ref, we_ref, be_ref, we1_ref, we2_ref, bee_ref,
                 ta_ref, tb_ref):
    h = nf_ref[:, :_H]
    x = nf_ref[:, _H:_H + 2]
    bf = jnp.bfloat16
    h_emb = jnp.dot(h.astype(bf), we_ref[...].astype(bf),
                    preferred_element_type=jnp.float32)
    h_emb = h_emb + be_ref[...][None, :]
    he16 = h_emb.astype(bf)
    a = jnp.dot(he16, we1_ref[...].astype(bf),
                preferred_element_type=jnp.float32)
    b = jnp.dot(he16, we2_ref[...].astype(bf),
                preferred_element_type=jnp.float32)
    b = b + bee_ref[...][None, :]
    pad = jnp.zeros((h.shape[0], _TW - _H - 2), jnp.float32)
    ta_ref[...] = jnp.concatenate([a, x, pad], axis=1)
    tb_ref[...] = jnp.concatenate([b, x, pad], axis=1)


def _build_tables(node_features, W_embed, b_embed, We1, We2, b_e):
    blk = 2000
    grid = _N // blk
    return pl.pallas_call(
        _tables_body,
        grid=(grid,),
        in_specs=[
            pl.BlockSpec((blk, _H + 2), lambda i: (i, 0)),
            pl.BlockSpec((_H, _HID), lambda i: (0, 0)),
            pl.BlockSpec((_HID,), lambda i: (0,)),
            pl.BlockSpec((_HID, _HID), lambda i: (0, 0)),
            pl.BlockSpec((_HID, _HID), lambda i: (0, 0)),
            pl.BlockSpec((_HID,), lambda i: (0,)),
        ],
        out_specs=[
            pl.BlockSpec((blk, _TW), lambda i: (i, 0)),
            pl.BlockSpec((blk, _TW), lambda i: (i, 0)),
        ],
        out_shape=[
            jax.ShapeDtypeStruct((_N, _TW), jnp.float32),
            jax.ShapeDtypeStruct((_N, _TW), jnp.float32),
        ],
    )(node_features, W_embed, b_embed, We1, We2, b_e)


def _edge_body(ta_hbm, tb_hbm, src_hbm, dst_hbm, w_hbm, wx_hbm,
               out_hbm,
               sall, dall, ra0, rb0, ra1, rb1, vacc, wv, wxv,
               sA0, sB0, sA1, sB1):
    cid = lax.axis_index("c")
    sid = lax.axis_index("s")
    wid = sid * _NCORES + cid

    def bf16round(x):
        i = lax.bitcast_convert_type(x, jnp.int32)
        r = i + 0x7FFF + ((i >> 16) & 1)
        r = r & jnp.int32(-65536)
        return lax.bitcast_convert_type(r, jnp.float32)

    iota = lax.broadcasted_iota(jnp.int32, (16,), 0)
    zero16 = jnp.zeros((16,), jnp.float32)
    pltpu.sync_copy(w_hbm, wv)
    pltpu.sync_copy(wx_hbm, wxv)
    wjs = [bf16round(wv[pl.ds(16 * j, 16)]) for j in range(8)]
    wxjs = [bf16round(wxv[pl.ds(16 * j, 16)]) for j in range(8)]

    def zero_body(k, carry):
        vacc[pl.ds(k * 16, 16)] = zero16
        return carry

    lax.fori_loop(0, (2 * _N) // 16, zero_body, 0)

    def take16(vec, idx):
        return jnp.take_along_axis(vec, idx, axis=0, mode="promise_in_bounds")

    base = wid * _EPW

    pltpu.sync_copy(src_hbm.at[pl.ds(base, _EPW)], sall)
    pltpu.sync_copy(dst_hbm.at[pl.ds(base, _EPW)], dall)

    def issue(ci, ra, rb, sA, sB):
        pltpu.async_copy(ta_hbm.at[sall.at[pl.ds(ci * _CH, _CH)]], ra, sA)
        pltpu.async_copy(tb_hbm.at[dall.at[pl.ds(ci * _CH, _CH)]], rb, sB)

    def wait(ra, rb, sA, sB):
        pltpu.make_async_copy(ta_hbm.at[pl.ds(0, _CH)], ra, sA).wait()
        pltpu.make_async_copy(tb_hbm.at[pl.ds(0, _CH)], rb, sB).wait()

    def compute(ci, ra, rb):
        @plsc.parallel_loop(0, _CH // 16)
        def group_body(g):
            rows = g * 16 + iota
            c0 = jnp.full((16,), _H, jnp.int32)
            c1 = jnp.full((16,), _H + 1, jnp.int32)
            xs0 = plsc.load_gather(ra, [rows, c0])
            xs1 = plsc.load_gather(ra, [rows, c1])
            xd0 = plsc.load_gather(rb, [rows, c0])
            xd1 = plsc.load_gather(rb, [rows, c1])
            dstv = dall[pl.ds(ci * _CH + g * 16, 16)]
            d0 = xs0 - xd0
            d1 = xs1 - xd1
            d2 = bf16round(d0 * d0 + d1 * d1)
            for l in range(16):
                e = g * 16 + l
                lidx = jnp.full((16,), l, jnp.int32)
                dist2 = take16(d2, lidx)
                acc = jnp.zeros((16,), jnp.float32)
                for j in range(8):
                    av = ra[e, pl.ds(16 * j, 16)]
                    bv = rb[e, pl.ds(16 * j, 16)]
                    m = jnp.maximum(bf16round(av + bv + dist2 * wjs[j]), 0.0)
                    acc = acc + m * wxjs[j]
                for sh in (8, 4, 2, 1):
                    acc = acc + take16(acc, iota ^ sh)
                dvec = jnp.where(iota == 0, take16(d0, lidx), take16(d1, lidx))
                didx = take16(dstv, lidx) + iota * _N
                plsc.addupdate_scatter(vacc, [didx], acc * dvec,
                                       mask=iota < 2)

    issue(0, ra0, rb0, sA0, sB0)

    def pair_body(k, carry):
        c0 = 2 * k
        wait(ra0, rb0, sA0, sB0)
        issue(c0 + 1, ra1, rb1, sA1, sB1)
        compute(c0, ra0, rb0)
        wait(ra1, rb1, sA1, sB1)
        issue(c0 + 2, ra0, rb0, sA0, sB0)
        compute(c0 + 1, ra1, rb1)
        return carry

    lax.fori_loop(0, (_NCHUNK - 1) // 2, pair_body, 0)
    wait(ra0, rb0, sA0, sB0)
    compute(_NCHUNK - 1, ra0, rb0)

    pltpu.sync_copy(vacc, out_hbm.at[wid])


def _edge_phase(ta, tb, src, dst, w_row, wx):
    mesh = plsc.VectorSubcoreMesh(core_axis_name="c", subcore_axis_name="s")
    f = pl.kernel(
        _edge_body,
        out_type=jax.ShapeDtypeStruct((_NW, 2 * _N), jnp.float32),
        mesh=mesh,
        scratch_types=[
            pltpu.VMEM((_EPW,), jnp.int32),
            pltpu.VMEM((_EPW,), jnp.int32),
            pltpu.VMEM((_CH, _TW), jnp.float32),
            pltpu.VMEM((_CH, _TW), jnp.float32),
            pltpu.VMEM((_CH, _TW), jnp.float32),
            pltpu.VMEM((_CH, _TW), jnp.float32),
            pltpu.VMEM((2 * _N,), jnp.float32),
            pltpu.VMEM((_HID,), jnp.float32),
            pltpu.VMEM((_HID,), jnp.float32),
            pltpu.SemaphoreType.DMA,
            pltpu.SemaphoreType.DMA,
            pltpu.SemaphoreType.DMA,
            pltpu.SemaphoreType.DMA,
        ],
        compiler_params=pltpu.CompilerParams(
            use_tc_tiling_on_sc=False, needs_layout_passes=False),
    )
    return f(ta, tb, src, dst, w_row, wx)


def _logits_body(p_ref, out_ref):
    v = jnp.sum(p_ref[...], axis=0)
    x = v[0:1, :]
    y = v[1:2, :]
    nx = -x
    c1 = (y > 0) & (y >= x) & (x > 0)
    c2 = (y > 0) & (x <= 0) & (y > nx)
    c3 = (y > 0) & (x < 0) & (y <= nx)
    c4 = (y <= 0) & (x < 0) & (y > x)
    c5 = (y < 0) & (x < 0) & (y <= x)
    c6 = (y < 0) & (x >= 0) & (-y > x)
    c7 = (y < 0) & (x > 0) & (-y <= x)
    cls = (c1.astype(jnp.int32) + 2 * c2.astype(jnp.int32)
           + 3 * c3.astype(jnp.int32) + 4 * c4.astype(jnp.int32)
           + 5 * c5.astype(jnp.int32) + 6 * c6.astype(jnp.int32)
           + 7 * c7.astype(jnp.int32))
    k = lax.broadcasted_iota(jnp.int32, (_NCLS, _N), 0)
    out_ref[...] = jnp.where(cls == k, 0.0, -1000.0)


def _make_logits(partials):
    return pl.pallas_call(
        _logits_body,
        out_shape=jax.ShapeDtypeStruct((_NCLS, _N), jnp.float32),
    )(partials)


def kernel(node_features, edge_index, W_embed, b_embed, W_e, b_e, W_h, b_h,
           W_x):
    del W_h, b_h
    We1 = W_e[:_HID]
    We2 = W_e[_HID:2 * _HID]
    w_row = W_e[2 * _HID]
    wx = W_x[:, 0]
    ta, tb = _build_tables(node_features, W_embed, b_embed, We1, We2, b_e)
    src = edge_index[0]
    dst = edge_index[1]
    partials = _edge_phase(ta, tb, src, dst, w_row, wx)
    logits_t = _make_logits(partials.reshape(_NW, 2, _N))
    return logits_t.T


# speedup vs baseline: 8.2040x; 1.0114x over previous
import functools

import jax
import jax.numpy as jnp
from jax import lax
from jax.experimental import pallas as pl
from jax.experimental.pallas import tpu as pltpu
from jax.experimental.pallas import tpu_sc as plsc

_N = 10000
_H = 128
_HID = 128
_E = 320000
_NCLS = 8

_NCORES = 2
_NSUB = 16
_NW = _NCORES * _NSUB
_EPW = _E // _NW
_CH = 80
_NCHUNK = _EPW // _CH
_TW = 144


def _tables_body(nf_ref, we_ref, be_ref, we1_ref, we2_ref, bee_ref,
                 ta_ref, tb_ref):
    h = nf_ref[:, :_H]
    x = nf_ref[:, _H:_H + 2]
    bf = jnp.bfloat16
    h_emb = jnp.dot(h.astype(bf), we_ref[...].astype(bf),
                    preferred_element_type=jnp.float32)
    h_emb = h_emb + be_ref[...][None, :]
    he16 = h_emb.astype(bf)
    a = jnp.dot(he16, we1_ref[...].astype(bf),
                preferred_element_type=jnp.float32)
    b = jnp.dot(he16, we2_ref[...].astype(bf),
                preferred_element_type=jnp.float32)
    b = b + bee_ref[...][None, :]
    pad = jnp.zeros((h.shape[0], _TW - _H - 2), jnp.float32)
    ta_ref[...] = jnp.concatenate([a, x, pad], axis=1)
    tb_ref[...] = jnp.concatenate([b, x, pad], axis=1)


def _build_tables(node_features, W_embed, b_embed, We1, We2, b_e):
    blk = 2000
    grid = _N // blk
    return pl.pallas_call(
        _tables_body,
        grid=(grid,),
        in_specs=[
            pl.BlockSpec((blk, _H + 2), lambda i: (i, 0)),
            pl.BlockSpec((_H, _HID), lambda i: (0, 0)),
            pl.BlockSpec((_HID,), lambda i: (0,)),
            pl.BlockSpec((_HID, _HID), lambda i: (0, 0)),
            pl.BlockSpec((_HID, _HID), lambda i: (0, 0)),
            pl.BlockSpec((_HID,), lambda i: (0,)),
        ],
        out_specs=[
            pl.BlockSpec((blk, _TW), lambda i: (i, 0)),
            pl.BlockSpec((blk, _TW), lambda i: (i, 0)),
        ],
        out_shape=[
            jax.ShapeDtypeStruct((_N, _TW), jnp.float32),
            jax.ShapeDtypeStruct((_N, _TW), jnp.float32),
        ],
    )(node_features, W_embed, b_embed, We1, We2, b_e)


def _edge_body(ta_hbm, tb_hbm, src_hbm, dst_hbm, w_hbm, wx_hbm,
               out_hbm,
               sall, dall, ra0, rb0, ra1, rb1, vacc, wv, wxv,
               sA0, sB0, sA1, sB1):
    cid = lax.axis_index("c")
    sid = lax.axis_index("s")
    wid = sid * _NCORES + cid

    def bf16round(x):
        i = lax.bitcast_convert_type(x, jnp.int32)
        r = i + 0x7FFF + ((i >> 16) & 1)
        r = r & jnp.int32(-65536)
        return lax.bitcast_convert_type(r, jnp.float32)

    iota = lax.broadcasted_iota(jnp.int32, (16,), 0)
    zero16 = jnp.zeros((16,), jnp.float32)
    pltpu.sync_copy(w_hbm, wv)
    pltpu.sync_copy(wx_hbm, wxv)
    wjs = [bf16round(wv[pl.ds(16 * j, 16)]) for j in range(8)]
    wxjs = [bf16round(wxv[pl.ds(16 * j, 16)]) for j in range(8)]

    def zero_body(k, carry):
        vacc[pl.ds(k * 16, 16)] = zero16
        return carry

    lax.fori_loop(0, (2 * _N) // 16, zero_body, 0)

    def take16(vec, idx):
        return jnp.take_along_axis(vec, idx, axis=0, mode="promise_in_bounds")

    base = wid * _EPW

    pltpu.sync_copy(src_hbm.at[pl.ds(base, _EPW)], sall)
    pltpu.sync_copy(dst_hbm.at[pl.ds(base, _EPW)], dall)

    def issue(ci, ra, rb, sA, sB):
        pltpu.async_copy(ta_hbm.at[sall.at[pl.ds(ci * _CH, _CH)]], ra, sA)
        pltpu.async_copy(tb_hbm.at[dall.at[pl.ds(ci * _CH, _CH)]], rb, sB)

    def wait(ra, rb, sA, sB):
        pltpu.make_async_copy(ta_hbm.at[pl.ds(0, _CH)], ra, sA).wait()
        pltpu.make_async_copy(tb_hbm.at[pl.ds(0, _CH)], rb, sB).wait()

    def compute(ci, ra, rb):
        @plsc.parallel_loop(0, _CH // 16)
        def group_body(g):
            rows = g * 16 + iota
            c0 = jnp.full((16,), _H, jnp.int32)
            c1 = jnp.full((16,), _H + 1, jnp.int32)
            xs0 = plsc.load_gather(ra, [rows, c0])
            xs1 = plsc.load_gather(ra, [rows, c1])
            xd0 = plsc.load_gather(rb, [rows, c0])
            xd1 = plsc.load_gather(rb, [rows, c1])
            dstv = dall[pl.ds(ci * _CH + g * 16, 16)]
            d0 = xs0 - xd0
            d1 = xs1 - xd1
            d2 = bf16round(d0 * d0 + d1 * d1)
            for l in range(16):
                e = g * 16 + l
                lidx = jnp.full((16,), l, jnp.int32)
                dist2 = take16(d2, lidx)
                accs = [jnp.zeros((16,), jnp.float32) for _ in range(2)]
                for j in range(8):
                    av = ra[e, pl.ds(16 * j, 16)]
                    bv = rb[e, pl.ds(16 * j, 16)]
                    m = jnp.maximum(bf16round(av + bv + dist2 * wjs[j]), 0.0)
                    accs[j % 2] = accs[j % 2] + m * wxjs[j]
                acc = accs[0] + accs[1]
                for sh in (8, 4, 2, 1):
                    acc = acc + take16(acc, iota ^ sh)
                dvec = jnp.where(iota == 0, take16(d0, lidx), take16(d1, lidx))
                didx = take16(dstv, lidx) + iota * _N
                plsc.addupdate_scatter(vacc, [didx], acc * dvec,
                                       mask=iota < 2)

    issue(0, ra0, rb0, sA0, sB0)

    def pair_body(k, carry):
        c0 = 2 * k
        wait(ra0, rb0, sA0, sB0)
        issue(c0 + 1, ra1, rb1, sA1, sB1)
        compute(c0, ra0, rb0)
        wait(ra1, rb1, sA1, sB1)
        issue(c0 + 2, ra0, rb0, sA0, sB0)
        compute(c0 + 1, ra1, rb1)
        return carry

    lax.fori_loop(0, (_NCHUNK - 1) // 2, pair_body, 0)
    wait(ra0, rb0, sA0, sB0)
    compute(_NCHUNK - 1, ra0, rb0)

    pltpu.sync_copy(vacc, out_hbm.at[wid])


def _edge_phase(ta, tb, src, dst, w_row, wx):
    mesh = plsc.VectorSubcoreMesh(core_axis_name="c", subcore_axis_name="s")
    f = pl.kernel(
        _edge_body,
        out_type=jax.ShapeDtypeStruct((_NW, 2 * _N), jnp.float32),
        mesh=mesh,
        scratch_types=[
            pltpu.VMEM((_EPW,), jnp.int32),
            pltpu.VMEM((_EPW,), jnp.int32),
            pltpu.VMEM((_CH, _TW), jnp.float32),
            pltpu.VMEM((_CH, _TW), jnp.float32),
            pltpu.VMEM((_CH, _TW), jnp.float32),
            pltpu.VMEM((_CH, _TW), jnp.float32),
            pltpu.VMEM((2 * _N,), jnp.float32),
            pltpu.VMEM((_HID,), jnp.float32),
            pltpu.VMEM((_HID,), jnp.float32),
            pltpu.SemaphoreType.DMA,
            pltpu.SemaphoreType.DMA,
            pltpu.SemaphoreType.DMA,
            pltpu.SemaphoreType.DMA,
        ],
        compiler_params=pltpu.CompilerParams(
            use_tc_tiling_on_sc=False, needs_layout_passes=False),
    )
    return f(ta, tb, src, dst, w_row, wx)


def _logits_body(p_ref, out_ref):
    v = jnp.sum(p_ref[...], axis=0)
    x = v[0:1, :]
    y = v[1:2, :]
    nx = -x
    c1 = (y > 0) & (y >= x) & (x > 0)
    c2 = (y > 0) & (x <= 0) & (y > nx)
    c3 = (y > 0) & (x < 0) & (y <= nx)
    c4 = (y <= 0) & (x < 0) & (y > x)
    c5 = (y < 0) & (x < 0) & (y <= x)
    c6 = (y < 0) & (x >= 0) & (-y > x)
    c7 = (y < 0) & (x > 0) & (-y <= x)
    cls = (c1.astype(jnp.int32) + 2 * c2.astype(jnp.int32)
           + 3 * c3.astype(jnp.int32) + 4 * c4.astype(jnp.int32)
           + 5 * c5.astype(jnp.int32) + 6 * c6.astype(jnp.int32)
           + 7 * c7.astype(jnp.int32))
    k = lax.broadcasted_iota(jnp.int32, (_NCLS, _N), 0)
    out_ref[...] = jnp.where(cls == k, 0.0, -1000.0)


def _make_logits(partials):
    return pl.pallas_call(
        _logits_body,
        out_shape=jax.ShapeDtypeStruct((_NCLS, _N), jnp.float32),
    )(partials)


def kernel(node_features, edge_index, W_embed, b_embed, W_e, b_e, W_h, b_h,
           W_x):
    del W_h, b_h
    We1 = W_e[:_HID]
    We2 = W_e[_HID:2 * _HID]
    w_row = W_e[2 * _HID]
    wx = W_x[:, 0]
    ta, tb = _build_tables(node_features, W_embed, b_embed, We1, We2, b_e)
    src = edge_index[0]
    dst = edge_index[1]
    partials = _edge_phase(ta, tb, src, dst, w_row, wx)
    logits_t = _make_logits(partials.reshape(_NW, 2, _N))
    return logits_t.T
